# baseline (speedup 1.0000x reference)
import jax
import jax.numpy as jnp
from jax.experimental import pallas as pl

_B, _T = 4, 8192
_DM, _KLAT, _DMODEL = 64, 16, 128
_KSEL, _KEFF = 8, 16
_NEG = -1e30
_GR, _GC = 64, 128


def _body(x_ref, fb_ref, wu_ref, bu_ref, wa_ref, ba_ref, m0c_ref, wma_ref,
          ws_ref, bs_ref, pos_ref, m0r_ref, wz_ref, bz_ref, wr_ref, br_ref,
          wh_ref, bh_ref, mu_ref, sig_ref, wl_ref, bl_ref, wp_ref, bp_ref,
          ycol_ref, tok_ref, mem_ref):
    f32 = jnp.float32
    wu = wu_ref[...]
    bu = bu_ref[...]
    wa = wa_ref[...]
    ba = ba_ref[...]
    ws = ws_ref[...]
    bs = bs_ref[...]
    pos = pos_ref[...]
    mwa = jnp.sum(m0c_ref[...] * wma_ref[...], axis=0, keepdims=True)
    m0r = m0r_ref[...]
    idxg = (_GR * jax.lax.broadcasted_iota(jnp.int32, (_GR, _GC), 1)
            + jax.lax.broadcasted_iota(jnp.int32, (_GR, _GC), 0))
    iota_l = jax.lax.broadcasted_iota(jnp.int32, (1, _T), 1)

    for b in range(_B):
        xb = x_ref[b]
        u = jnp.tanh(jnp.dot(xb, wu, preferred_element_type=f32) + bu)
        a = jnp.tanh(jnp.dot(u, wa, preferred_element_type=f32) + mwa + ba)
        s_col = jnp.sum(a * ws, axis=1, keepdims=True) + bs + pos
        sg = jnp.concatenate(
            [s_col[i * _GR:(i + 1) * _GR, :] for i in range(_GC)], axis=1)
        maxs = jnp.max(sg, keepdims=True)
        sumexp = jnp.sum(jnp.exp(sg - maxs), keepdims=True)

        ms = sg
        yst = jnp.zeros((_GR, _GC), f32)
        picks = []
        for _ in range(_KSEL):
            v = jnp.max(ms, keepdims=True)
            pidx = jnp.min(jnp.where(ms == v, idxg, _T), keepdims=True)
            pj = jnp.exp(v - maxs) / sumexp
            yst = yst + pj * (idxg == pidx).astype(f32)
            ms = jnp.where(jnp.abs(idxg - pidx) <= 1, _NEG, ms)
            picks.append((pidx, pj))
        sump = jnp.sum(yst, keepdims=True)

        yw = yst
        tvals, tis = [], []
        for _ in range(_KEFF):
            v = jnp.max(yw, keepdims=True)
            ti = jnp.min(jnp.where(yw == v, idxg, _T), keepdims=True)
            tvals.append(v)
            tis.append(ti)
            yw = jnp.where(idxg == ti, -1.0, yw)

        ycol_ref[:, b:b + 1] = jnp.concatenate(
            [yst[:, i:i + 1] for i in range(_GC)], axis=0)

        onehot = jnp.concatenate(
            [(iota_l == t).astype(f32) for t in tis], axis=0)
        xg = jnp.dot(onehot, xb, preferred_element_type=f32)
        tv = jnp.concatenate(tvals, axis=0)
        tii = jnp.concatenate(tis, axis=0)

        cn = jnp.zeros((_KEFF, 1), f32)
        mean_cn_num = jnp.zeros((1, 1), f32)
        for pidx, pj in picks:
            cn = cn + pj * (pidx <= tii).astype(f32)
            mean_cn_num = mean_cn_num + pj * (_T - pidx).astype(f32)
        denom = sump + 1e-8
        cn = cn / denom
        mean_cn = mean_cn_num / (denom * _T)

        xmean = jnp.mean(xb, axis=0, keepdims=True)
        posn = tii.astype(f32) * (1.0 / _T)
        dvec = jnp.concatenate([xg, tv, posn, cn], axis=1)
        mp = jnp.full((1, 1), (_T - 1) / (2.0 * _T), f32)
        dmean = jnp.concatenate(
            [xmean, sump * (1.0 / _T), mp, mean_cn], axis=1)
        c = dvec - dmean
        c = c / (jnp.sqrt(jnp.sum(c * c, axis=1, keepdims=True)) + 1e-6)
        zz = (c - mu_ref[...]) / sig_ref[...]
        lif = jnp.tanh(jnp.dot(zz, wl_ref[...], preferred_element_type=f32)
                       + bl_ref[...])
        lif = lif / (jnp.sqrt(jnp.sum(lif * lif, axis=1, keepdims=True)) + 1e-6)
        tok_ref[b] = (jnp.dot(lif, wp_ref[...], preferred_element_type=f32)
                      + bp_ref[...])

        u8 = jnp.tanh(jnp.dot(xg[0:_KSEL, :], wu, preferred_element_type=f32)
                      + bu)
        w8 = tv[0:_KSEL, :] / denom
        ctx = jnp.sum(w8 * u8, axis=0, keepdims=True)
        inp = jnp.concatenate([ctx, fb_ref[b:b + 1, :]], axis=1)
        xh = jnp.concatenate([inp, m0r], axis=1)
        zg = jax.nn.sigmoid(jnp.dot(xh, wz_ref[...], preferred_element_type=f32)
                            + bz_ref[...])
        rg = jax.nn.sigmoid(jnp.dot(xh, wr_ref[...], preferred_element_type=f32)
                            + br_ref[...])
        xrh = jnp.concatenate([inp, rg * m0r], axis=1)
        hh = jnp.tanh(jnp.dot(xrh, wh_ref[...], preferred_element_type=f32)
                      + bh_ref[...])
        m1 = (1.0 - zg) * m0r + zg * hh
        mem_ref[b] = jnp.concatenate([m0r, m1], axis=0)


def kernel(x, feedback, params):
    p = params
    B, T, _ = x.shape
    f32 = jnp.float32
    args = (
        x, feedback,
        p['W_u'], p['b_u'].reshape(1, -1),
        p['W_a'], p['b_a'].reshape(1, -1),
        p['m0'].reshape(-1, 1), p['W_ma'],
        p['w_s'].reshape(1, -1), p['b_s'].reshape(1, 1),
        p['pos_bias'][:T].reshape(-1, 1), p['m0'].reshape(1, -1),
        p['W_z'], p['b_z'].reshape(1, -1),
        p['W_r'], p['b_r'].reshape(1, -1),
        p['W_h'], p['b_h'].reshape(1, -1),
        p['mu'].reshape(1, -1), p['sigma'].reshape(1, -1),
        p['W_lift'], p['b_lift'].reshape(1, -1),
        p['W_proj'], p['b_proj'].reshape(1, -1),
    )
    ycol, tokens, mem = pl.pallas_call(
        _body,
        out_shape=(
            jax.ShapeDtypeStruct((T, B), f32),
            jax.ShapeDtypeStruct((B, _KEFF, _DMODEL), f32),
            jax.ShapeDtypeStruct((B, 2, _DM), f32),
        ),
    )(*args)
    y_star = ycol.T
    all_y = y_star[:, None, :]
    return tokens, y_star, all_y, mem


# speedup vs baseline: 2.1041x; 2.1041x over previous
import jax
import jax.numpy as jnp
from jax.experimental import pallas as pl

_B, _T = 4, 8192
_DM, _KLAT, _DMODEL = 64, 16, 128
_KSEL, _KEFF = 8, 16
_NEG = -1e30


def _body(x_ref, fb_ref, wu_ref, bur_ref, buc_ref, wa_ref, bac_ref, wmat_ref,
          m0r_ref, wsc_ref, bs_ref, pos_ref, wz_ref, bz_ref, wr_ref, br_ref,
          wh_ref, bh_ref, mu_ref, sig_ref, wl_ref, bl_ref, wp_ref, bp_ref,
          y_ref, tok_ref, mem_ref):
    f32 = jnp.float32
    i32 = jnp.int32
    wu = wu_ref[...]
    bur = bur_ref[...]
    buc = buc_ref[...]
    wa = wa_ref[...]
    bac = bac_ref[...]
    wsc = wsc_ref[...]
    bs = bs_ref[...]
    pos = pos_ref[...]
    m0r = m0r_ref[...]
    mwa_c = jnp.sum(wmat_ref[...] * m0r, axis=1, keepdims=True)
    iota_l = jax.lax.broadcasted_iota(i32, (1, _T), 1)
    ones_row = jnp.ones((1, _T), f32)

    dn_t = (((0,), (1,)), ((), ()))
    dn_tt = (((0,), (0,)), ((), ()))

    for b in range(_B):
        xb = x_ref[b]
        ut = jnp.tanh(
            jax.lax.dot_general(wu, xb, dn_t, preferred_element_type=f32)
            + buc)
        at = jnp.tanh(
            jax.lax.dot_general(wa, ut, dn_tt, preferred_element_type=f32)
            + mwa_c + bac)
        s = jnp.sum(at * wsc, axis=0, keepdims=True) + bs + pos
        maxs = jnp.max(s, keepdims=True)
        sumexp = jnp.sum(jnp.exp(s - maxs), keepdims=True)

        ms = s
        pidxs, pjs = [], []
        for _ in range(_KSEL):
            v = jnp.max(ms, keepdims=True)
            pidx = jnp.min(jnp.where(ms == v, iota_l, _T), keepdims=True)
            pjs.append(jnp.exp(v - maxs) / sumexp)
            pidxs.append(pidx)
            ms = jnp.where(jnp.abs(iota_l - pidx) <= 1, _NEG, ms)

        p8r = jnp.concatenate(pjs, axis=1)
        i8r = jnp.concatenate(pidxs, axis=1)
        p8c = jnp.concatenate(pjs, axis=0)
        i8c = jnp.concatenate(pidxs, axis=0)

        y_row = jnp.zeros((1, _T), f32)
        for pidx, pj in zip(pidxs, pjs):
            y_row = y_row + pj * (iota_l == pidx).astype(f32)
        y_ref[b:b + 1, :] = y_row
        sump = jnp.sum(p8r, keepdims=True)
        denom = sump + 1e-8

        before = (p8c > p8r) | ((p8c == p8r) & (i8c < i8r))
        rank = jnp.sum(before.astype(i32), axis=0, keepdims=True)
        k8c = jax.lax.broadcasted_iota(i32, (_KSEL, 1), 0)
        mrank = (rank == k8c).astype(f32)
        svals = jnp.sum(mrank * p8r, axis=1, keepdims=True)
        sidx = jnp.sum(mrank.astype(i32) * i8r, axis=1, keepdims=True)
        t16r = jax.lax.broadcasted_iota(i32, (1, 2 * _KSEL), 1)
        picked = jnp.zeros((1, 2 * _KSEL), jnp.bool_)
        for pidx in pidxs:
            picked = picked | (t16r == pidx)
        free = ~picked
        t16c = jax.lax.broadcasted_iota(i32, (2 * _KSEL, 1), 0)
        free_c = jnp.sum((t16c == t16r).astype(i32)
                         * free.astype(i32), axis=1, keepdims=True)
        bc = jnp.sum(jnp.where((t16c < t16r) & (free_c > 0), 1, 0),
                     axis=0, keepdims=True)
        m2 = ((bc == k8c) & free).astype(i32)
        zidx = jnp.sum(m2 * t16r, axis=1, keepdims=True)
        tii = jnp.concatenate([sidx, zidx], axis=0)
        tv = jnp.concatenate([svals, jnp.zeros((_KSEL, 1), f32)], axis=0)

        onehot = (tii == iota_l).astype(f32)
        xg = jnp.dot(onehot, xb, preferred_element_type=f32)
        xmean = jnp.dot(ones_row, xb,
                        preferred_element_type=f32) * (1.0 / _T)

        i8f = i8r.astype(f32)
        cn = jnp.sum(p8r * (i8r <= tii).astype(f32), axis=1,
                     keepdims=True) / denom
        mean_cn = jnp.sum(p8r * (_T - i8f), keepdims=True) / (denom * _T)

        posn = tii.astype(f32) * (1.0 / _T)
        dvec = jnp.concatenate([xg, tv, posn, cn], axis=1)
        mp = jnp.full((1, 1), (_T - 1) / (2.0 * _T), f32)
        dmean = jnp.concatenate(
            [xmean, sump * (1.0 / _T), mp, mean_cn], axis=1)
        c = dvec - dmean
        c = c / (jnp.sqrt(jnp.sum(c * c, axis=1, keepdims=True)) + 1e-6)
        zz = (c - mu_ref[...]) / sig_ref[...]
        lif = jnp.tanh(jnp.dot(zz, wl_ref[...], preferred_element_type=f32)
                       + bl_ref[...])
        lif = lif / (jnp.sqrt(jnp.sum(lif * lif, axis=1, keepdims=True)) + 1e-6)
        tok_ref[b] = (jnp.dot(lif, wp_ref[...], preferred_element_type=f32)
                      + bp_ref[...])

        u8 = jnp.tanh(jnp.dot(xg[0:_KSEL, :], wu, preferred_element_type=f32)
                      + bur)
        w8 = tv[0:_KSEL, :] / denom
        ctx = jnp.sum(w8 * u8, axis=0, keepdims=True)
        inp = jnp.concatenate([ctx, fb_ref[b:b + 1, :]], axis=1)
        xh = jnp.concatenate([inp, m0r], axis=1)
        zg = jax.nn.sigmoid(jnp.dot(xh, wz_ref[...], preferred_element_type=f32)
                            + bz_ref[...])
        rg = jax.nn.sigmoid(jnp.dot(xh, wr_ref[...], preferred_element_type=f32)
                            + br_ref[...])
        xrh = jnp.concatenate([inp, rg * m0r], axis=1)
        hh = jnp.tanh(jnp.dot(xrh, wh_ref[...], preferred_element_type=f32)
                      + bh_ref[...])
        m1 = (1.0 - zg) * m0r + zg * hh
        mem_ref[b] = jnp.concatenate([m0r, m1], axis=0)


def kernel(x, feedback, params):
    p = params
    B, T, _ = x.shape
    f32 = jnp.float32
    args = (
        x, feedback,
        p['W_u'], p['b_u'].reshape(1, -1), p['b_u'].reshape(-1, 1),
        p['W_a'], p['b_a'].reshape(-1, 1), p['W_ma'].T,
        p['m0'].reshape(1, -1),
        p['w_s'].reshape(-1, 1), p['b_s'].reshape(1, 1),
        p['pos_bias'][:T].reshape(1, -1),
        p['W_z'], p['b_z'].reshape(1, -1),
        p['W_r'], p['b_r'].reshape(1, -1),
        p['W_h'], p['b_h'].reshape(1, -1),
        p['mu'].reshape(1, -1), p['sigma'].reshape(1, -1),
        p['W_lift'], p['b_lift'].reshape(1, -1),
        p['W_proj'], p['b_proj'].reshape(1, -1),
    )
    y_star, tokens, mem = pl.pallas_call(
        _body,
        out_shape=(
            jax.ShapeDtypeStruct((B, T), f32),
            jax.ShapeDtypeStruct((B, _KEFF, _DMODEL), f32),
            jax.ShapeDtypeStruct((B, 2, _DM), f32),
        ),
    )(*args)
    all_y = y_star[:, None, :]
    return tokens, y_star, all_y, mem
